# baseline (speedup 1.0000x reference)
import functools

import jax
import jax.numpy as jnp
from jax import lax
from jax.experimental import pallas as pl
from jax.experimental.pallas import tpu as pltpu
from jax.experimental.pallas import tpu_sc as plsc

_B = 4096
_CTX = 20
_D = 16
_V = 100000

_NC = 2
_NS = 16
_NW = _NC * _NS
_BPW = _B // _NW

_BV = 512


_BH = _B // _NC
_G = 512
_NG = _BH // _G


def _gather_mean_body(
    idx_hbm, tablet_hbm, avgt_hbm, row_v, idx_v, out_v, sem_r, sem0, sem1
):
    d = lax.axis_index("s")
    h = lax.axis_index("c")
    base = h * _BH
    sems = [sem0, sem1]
    row_cp = pltpu.async_copy(tablet_hbm.at[d], row_v, sem_r)

    def idx_start(g):
        return pltpu.async_copy(
            idx_hbm.at[:, pl.ds(base + g * _G, _G)], idx_v.at[g % 2], sems[g % 2]
        )

    cps = {0: idx_start(0)}
    row_cp.wait()
    for g in range(_NG):
        if g + 1 < _NG:
            cps[g + 1] = idx_start(g + 1)
        cps[g].wait()
        buf = g % 2

        def bb_body(bb, carry, _g=g, _buf=buf):
            acc = plsc.load_gather(row_v, [idx_v[_buf, 0, pl.ds(bb * 16, 16)]])
            for c in range(1, _CTX):
                acc = acc + plsc.load_gather(
                    row_v, [idx_v[_buf, c, pl.ds(bb * 16, 16)]]
                )
            out_off = _g * _G + bb * 16
            out_v[pl.ds(out_off, 16)] = acc * (1.0 / _CTX)
            return carry

        lax.fori_loop(0, _G // 16, bb_body, 0)
    pltpu.sync_copy(out_v, avgt_hbm.at[d, pl.ds(base, _BH)])


_gather_mean = functools.partial(
    pl.kernel,
    out_type=jax.ShapeDtypeStruct((_D, _B), jnp.float32),
    mesh=plsc.VectorSubcoreMesh(core_axis_name="c", subcore_axis_name="s"),
    scratch_types=[
        pltpu.VMEM((_V,), jnp.float32),
        pltpu.VMEM((2, _CTX, _G), jnp.int32),
        pltpu.VMEM((_BH,), jnp.float32),
        pltpu.SemaphoreType.DMA,
        pltpu.SemaphoreType.DMA,
        pltpu.SemaphoreType.DMA,
    ],
    compiler_params=pltpu.CompilerParams(
        use_tc_tiling_on_sc=True, needs_layout_passes=False
    ),
)(_gather_mean_body)


def _matmul_body(we_ref, ae_ref, out_ref):
    out_ref[...] = lax.dot_general(
        we_ref[...],
        ae_ref[...],
        (((0,), (0,)), ((), ())),
        preferred_element_type=jnp.float32,
    )


def kernel(context_words, emb_table, W, b):
    idx_t = context_words.astype(jnp.int32).T

    avg_t = _gather_mean(idx_t, emb_table.T)

    w_ext = jnp.concatenate([W.T, b[None, :]], axis=0)
    avg_ext = jnp.concatenate(
        [avg_t, jnp.ones((1, _B), jnp.float32)], axis=0
    )

    out_t = pl.pallas_call(
        _matmul_body,
        grid=(pl.cdiv(_V, _BV),),
        in_specs=[
            pl.BlockSpec((_D + 1, _BV), lambda i: (0, i)),
            pl.BlockSpec((_D + 1, _B), lambda i: (0, 0)),
        ],
        out_specs=pl.BlockSpec((_BV, _B), lambda i: (i, 0)),
        out_shape=jax.ShapeDtypeStruct((_V, _B), jnp.float32),
        compiler_params=pltpu.CompilerParams(
            dimension_semantics=("arbitrary",),
            vmem_limit_bytes=100 * 1024 * 1024,
        ),
    )(w_ext, avg_ext)
    return out_t.T


# speedup vs baseline: 1.0032x; 1.0032x over previous
import functools

import jax
import jax.numpy as jnp
from jax import lax
from jax.experimental import pallas as pl
from jax.experimental.pallas import tpu as pltpu
from jax.experimental.pallas import tpu_sc as plsc

_B = 4096
_CTX = 20
_D = 16
_V = 100000

_NC = 2
_NS = 16
_NW = _NC * _NS
_BPW = _B // _NW

_BV = 1024


_BH = _B // _NC
_G = 512
_NG = _BH // _G


def _gather_mean_body(
    idx_hbm, tablet_hbm, avgt_hbm, row_v, idx_v, out_v, sem_r, sem0, sem1
):
    d = lax.axis_index("s")
    h = lax.axis_index("c")
    base = h * _BH
    sems = [sem0, sem1]
    row_cp = pltpu.async_copy(tablet_hbm.at[d], row_v, sem_r)

    def idx_start(g):
        return pltpu.async_copy(
            idx_hbm.at[:, pl.ds(base + g * _G, _G)], idx_v.at[g % 2], sems[g % 2]
        )

    cps = {0: idx_start(0)}
    row_cp.wait()
    for g in range(_NG):
        if g + 1 < _NG:
            cps[g + 1] = idx_start(g + 1)
        cps[g].wait()
        buf = g % 2

        def bb_body(bb, carry, _g=g, _buf=buf):
            acc = plsc.load_gather(row_v, [idx_v[_buf, 0, pl.ds(bb * 16, 16)]])
            for c in range(1, _CTX):
                acc = acc + plsc.load_gather(
                    row_v, [idx_v[_buf, c, pl.ds(bb * 16, 16)]]
                )
            out_off = _g * _G + bb * 16
            out_v[pl.ds(out_off, 16)] = acc * (1.0 / _CTX)
            return carry

        lax.fori_loop(0, _G // 16, bb_body, 0)
    pltpu.sync_copy(out_v, avgt_hbm.at[d, pl.ds(base, _BH)])


_gather_mean = functools.partial(
    pl.kernel,
    out_type=jax.ShapeDtypeStruct((_D, _B), jnp.float32),
    mesh=plsc.VectorSubcoreMesh(core_axis_name="c", subcore_axis_name="s"),
    scratch_types=[
        pltpu.VMEM((_V,), jnp.float32),
        pltpu.VMEM((2, _CTX, _G), jnp.int32),
        pltpu.VMEM((_BH,), jnp.float32),
        pltpu.SemaphoreType.DMA,
        pltpu.SemaphoreType.DMA,
        pltpu.SemaphoreType.DMA,
    ],
    compiler_params=pltpu.CompilerParams(
        use_tc_tiling_on_sc=True, needs_layout_passes=False
    ),
)(_gather_mean_body)


def _matmul_body(we_ref, ae_ref, out_ref):
    out_ref[...] = lax.dot_general(
        we_ref[...],
        ae_ref[...],
        (((0,), (0,)), ((), ())),
        preferred_element_type=jnp.float32,
    )


def kernel(context_words, emb_table, W, b):
    idx_t = context_words.astype(jnp.int32).T

    avg_t = _gather_mean(idx_t, emb_table.T)

    w_ext = jnp.concatenate([W.T, b[None, :]], axis=0)
    avg_ext = jnp.concatenate(
        [avg_t, jnp.ones((1, _B), jnp.float32)], axis=0
    )

    out_t = pl.pallas_call(
        _matmul_body,
        grid=(pl.cdiv(_V, _BV),),
        in_specs=[
            pl.BlockSpec((_D + 1, _BV), lambda i: (0, i)),
            pl.BlockSpec((_D + 1, _B), lambda i: (0, 0)),
        ],
        out_specs=pl.BlockSpec((_BV, _B), lambda i: (i, 0)),
        out_shape=jax.ShapeDtypeStruct((_V, _B), jnp.float32),
        compiler_params=pltpu.CompilerParams(
            dimension_semantics=("parallel",),
            vmem_limit_bytes=100 * 1024 * 1024,
        ),
    )(w_ext, avg_ext)
    return out_t.T
